# Initial kernel scaffold; baseline (speedup 1.0000x reference)
import jax
import jax.numpy as jnp
from jax.experimental import pallas as pl


def kernel(inputs, kernel):
    raise NotImplementedError("write your pallas kernel here")


# speedup vs baseline: 8.4006x; 8.4006x over previous
import functools

import jax
import jax.numpy as jnp
from jax import lax
from jax.experimental import pallas as pl
from jax.experimental.pallas import tpu as pltpu
from jax.experimental.pallas import tpu_sc as plsc

_VOCAB = 100000
_D = 4 * 4 * 16
_B = 4096 * 26
_NC = 2
_NS = 16
_NW = _NC * _NS
_B_PER_W = _B // _NW
_CHUNK = 128
_NCHUNK = _B_PER_W // _CHUNK

_mesh = plsc.VectorSubcoreMesh(core_axis_name="c", subcore_axis_name="s")


@functools.partial(
    pl.kernel,
    mesh=_mesh,
    out_type=jax.ShapeDtypeStruct((_B, _D), jnp.float32),
    scratch_types=[
        pltpu.VMEM((_NCHUNK, _CHUNK), jnp.int32),
        pltpu.VMEM((_CHUNK, _D), jnp.float32),
        pltpu.VMEM((_CHUNK, _D), jnp.float32),
        pltpu.SemaphoreType.DMA,
        pltpu.SemaphoreType.DMA,
    ],
)
def _sc_gather(idx_hbm, table_hbm, out_hbm, idx_v, rows0, rows1, g0, g1):
    cid = lax.axis_index("c")
    sid = lax.axis_index("s")
    wid = sid * _NC + cid
    base = wid * _B_PER_W

    pltpu.sync_copy(idx_hbm.at[wid], idx_v)

    pltpu.async_copy(table_hbm.at[idx_v.at[0]], rows0, g0)
    pltpu.async_copy(table_hbm.at[idx_v.at[1]], rows1, g1)

    def _wait_gather(rows, sem):
        pltpu.make_async_copy(table_hbm.at[idx_v.at[0]], rows, sem).wait()

    def body(k, carry):
        c = 2 * k
        _wait_gather(rows0, g0)
        pltpu.sync_copy(rows0, out_hbm.at[pl.ds(base + c * _CHUNK, _CHUNK)])
        pltpu.async_copy(table_hbm.at[idx_v.at[c + 2]], rows0, g0)
        _wait_gather(rows1, g1)
        pltpu.sync_copy(rows1, out_hbm.at[pl.ds(base + (c + 1) * _CHUNK, _CHUNK)])
        pltpu.async_copy(table_hbm.at[idx_v.at[c + 3]], rows1, g1)
        return carry

    lax.fori_loop(0, _NCHUNK // 2 - 1, body, 0)

    last = _NCHUNK - 2
    _wait_gather(rows0, g0)
    pltpu.sync_copy(rows0, out_hbm.at[pl.ds(base + last * _CHUNK, _CHUNK)])
    _wait_gather(rows1, g1)
    pltpu.sync_copy(rows1, out_hbm.at[pl.ds(base + (last + 1) * _CHUNK, _CHUNK)])


def kernel(inputs, kernel):
    table = kernel.reshape(_VOCAB, _D)
    idx = inputs.reshape(_NW, _NCHUNK, _CHUNK)
    out = _sc_gather(idx, table)
    return out.reshape(inputs.shape + kernel.shape[1:])
